# Initial kernel scaffold; baseline (speedup 1.0000x reference)
import functools

import jax
import jax.numpy as jnp
from jax import lax
from jax.experimental import pallas as pl
from jax.experimental.pallas import tpu as pltpu
from jax.experimental.pallas import tpu_sc as plsc

VOCAB = 1000000
DIM = 64
B = 16384
NEG = 20

NUM_CORES = 2
NUM_SUBCORES = 16
NW = NUM_CORES * NUM_SUBCORES
BPW = B // NW
CHUNK = 32
NCHUNK = BPW // CHUNK
NEGC = CHUNK * NEG
IDX_BLK = 128


def _sc_scores_kernel(tgt_hbm, ctx_hbm, negidx_hbm, in_emb, out_emb,
                      pos_out, negs_out,
                      tgt_v, ctx_v, neg_v, vrows, urows, nrows,
                      pos_s, neg_s, sem):
    wid = lax.axis_index("s") * NUM_CORES + lax.axis_index("c")

    def chunk_body(c, _):
        base = wid * BPW + c * CHUNK
        pltpu.sync_copy(tgt_hbm.at[pl.ds(base, CHUNK)], tgt_v)
        pltpu.sync_copy(ctx_hbm.at[pl.ds(base, CHUNK)], ctx_v)
        pltpu.sync_copy(negidx_hbm.at[pl.ds(base * NEG, NEGC)], neg_v)
        cps = [
            pltpu.async_copy(in_emb.at[tgt_v], vrows, sem),
            pltpu.async_copy(out_emb.at[ctx_v], urows, sem),
        ]
        for j in range(NEGC // IDX_BLK):
            cps.append(pltpu.async_copy(
                out_emb.at[neg_v.at[pl.ds(j * IDX_BLK, IDX_BLK)]],
                nrows.at[pl.ds(j * IDX_BLK, IDX_BLK)], sem))
        for cp in cps:
            cp.wait()

        def b_body(b, _):
            vv = [vrows[b, pl.ds(16 * j, 16)] for j in range(4)]
            acc = urows[b, pl.ds(0, 16)] * vv[0]
            for j in range(1, 4):
                acc = acc + urows[b, pl.ds(16 * j, 16)] * vv[j]
            pos_s[b] = jnp.sum(acc)
            for n in range(NEG):
                i = b * NEG + n
                nacc = nrows[i, pl.ds(0, 16)] * vv[0]
                for j in range(1, 4):
                    nacc = nacc + nrows[i, pl.ds(16 * j, 16)] * vv[j]
                neg_s[i] = jnp.sum(nacc)
            return 0

        lax.fori_loop(0, CHUNK, b_body, 0)
        pltpu.sync_copy(pos_s, pos_out.at[pl.ds(base, CHUNK)])
        pltpu.sync_copy(neg_s, negs_out.at[pl.ds(base * NEG, NEGC)])
        return 0

    lax.fori_loop(0, NCHUNK, chunk_body, 0)


@functools.partial(
    pl.kernel,
    mesh=plsc.VectorSubcoreMesh(core_axis_name="c", subcore_axis_name="s"),
    out_type=[
        jax.ShapeDtypeStruct((B,), jnp.float32),
        jax.ShapeDtypeStruct((B * NEG,), jnp.float32),
    ],
    scratch_types=[
        pltpu.VMEM((CHUNK,), jnp.int32),
        pltpu.VMEM((CHUNK,), jnp.int32),
        pltpu.VMEM((NEGC,), jnp.int32),
        pltpu.VMEM((CHUNK, DIM), jnp.float32),
        pltpu.VMEM((CHUNK, DIM), jnp.float32),
        pltpu.VMEM((NEGC, DIM), jnp.float32),
        pltpu.VMEM((CHUNK,), jnp.float32),
        pltpu.VMEM((NEGC,), jnp.float32),
        pltpu.SemaphoreType.DMA,
    ],
)
def _sc_scores(*args):
    _sc_scores_kernel(*args)


def _tc_loss_body(pos_ref, neg_ref, out_ref):
    pos = pos_ref[...]
    neg = neg_ref[...]
    total = jnp.sum(jax.nn.log_sigmoid(pos)) + jnp.sum(jax.nn.log_sigmoid(-neg))
    out_ref[0, 0] = -total / B


def kernel(target_input, context, neg, input_emb, output_emb):
    tgt = target_input.astype(jnp.int32)
    ctx = context.astype(jnp.int32)
    negidx = neg.astype(jnp.int32).reshape(-1)
    pos_sc, negs_sc = _sc_scores(tgt, ctx, negidx, input_emb, output_emb)
    loss = pl.pallas_call(
        _tc_loss_body,
        out_shape=jax.ShapeDtypeStruct((1, 1), jnp.float32),
        out_specs=pl.BlockSpec(memory_space=pltpu.SMEM),
    )(pos_sc.reshape(B // 128, 128), negs_sc.reshape(B * NEG // 128, 128))
    return loss[0, 0]


# speedup vs baseline: 5.1885x; 5.1885x over previous
import functools

import jax
import jax.numpy as jnp
from jax import lax
from jax.experimental import pallas as pl
from jax.experimental.pallas import tpu as pltpu
from jax.experimental.pallas import tpu_sc as plsc

VOCAB = 1000000
DIM = 64
B = 16384
NEG = 20
NSCORE = 32

NUM_CORES = 2
NUM_SUBCORES = 16
NW = NUM_CORES * NUM_SUBCORES
BPW = B // NW
CHUNK = 32
NCHUNK = BPW // CHUNK
NEGC = CHUNK * NEG
IDX_BLK = 128


def _sc_scores_kernel(tgt_hbm, ctx_hbm, negidx_hbm, in_emb, out_emb,
                      scores_out,
                      tgt_v, ctx_v, neg_v, vrows, urows, nrows,
                      scores_s, sem):
    wid = lax.axis_index("s") * NUM_CORES + lax.axis_index("c")
    lanes = lax.iota(jnp.int32, 16)

    def chunk_body(c, _):
        base = wid * BPW + c * CHUNK
        pltpu.sync_copy(tgt_hbm.at[pl.ds(base, CHUNK)], tgt_v)
        pltpu.sync_copy(ctx_hbm.at[pl.ds(base, CHUNK)], ctx_v)
        pltpu.sync_copy(negidx_hbm.at[pl.ds(base * NEG, NEGC)], neg_v)
        cps = [
            pltpu.async_copy(in_emb.at[tgt_v], vrows, sem),
            pltpu.async_copy(out_emb.at[ctx_v], urows, sem),
        ]
        for j in range(NEGC // IDX_BLK):
            cps.append(pltpu.async_copy(
                out_emb.at[neg_v.at[pl.ds(j * IDX_BLK, IDX_BLK)]],
                nrows.at[pl.ds(j * IDX_BLK, IDX_BLK)], sem))
        for cp in cps:
            cp.wait()

        def b_body(b, _):
            vv = [vrows[b, pl.ds(16 * j, 16)] for j in range(4)]

            def dot_rows(rows_ref, i):
                acc = rows_ref[i, pl.ds(0, 16)] * vv[0]
                for j in range(1, 4):
                    acc = acc + rows_ref[i, pl.ds(16 * j, 16)] * vv[j]
                return jnp.sum(acc)

            vec0 = jnp.zeros((16,), jnp.float32)
            vec0 = jnp.where(lanes == 0, dot_rows(urows, b), vec0)
            for n in range(15):
                s = dot_rows(nrows, b * NEG + n)
                vec0 = jnp.where(lanes == n + 1, s, vec0)
            vec1 = jnp.zeros((16,), jnp.float32)
            for n in range(15, NEG):
                s = dot_rows(nrows, b * NEG + n)
                vec1 = jnp.where(lanes == n - 15, s, vec1)
            scores_s[b, pl.ds(0, 16)] = vec0
            scores_s[b, pl.ds(16, 16)] = vec1
            return 0

        lax.fori_loop(0, CHUNK, b_body, 0)
        pltpu.sync_copy(scores_s, scores_out.at[pl.ds(base, CHUNK)])
        return 0

    lax.fori_loop(0, NCHUNK, chunk_body, 0)


@functools.partial(
    pl.kernel,
    mesh=plsc.VectorSubcoreMesh(core_axis_name="c", subcore_axis_name="s"),
    compiler_params=pltpu.CompilerParams(
        needs_layout_passes=False, use_tc_tiling_on_sc=False),
    out_type=jax.ShapeDtypeStruct((B, NSCORE), jnp.float32),
    scratch_types=[
        pltpu.VMEM((CHUNK,), jnp.int32),
        pltpu.VMEM((CHUNK,), jnp.int32),
        pltpu.VMEM((NEGC,), jnp.int32),
        pltpu.VMEM((CHUNK, DIM), jnp.float32),
        pltpu.VMEM((CHUNK, DIM), jnp.float32),
        pltpu.VMEM((NEGC, DIM), jnp.float32),
        pltpu.VMEM((CHUNK, NSCORE), jnp.float32),
        pltpu.SemaphoreType.DMA,
    ],
)
def _sc_scores(*args):
    _sc_scores_kernel(*args)


def _tc_loss_body(scores_ref, out_ref):
    x = scores_ref[...]
    col = lax.broadcasted_iota(jnp.int32, x.shape, 1)
    sign = jnp.where(col == 0, 1.0, -1.0)
    valid = col <= NEG
    ls = jax.nn.log_sigmoid(x * sign)
    total = jnp.sum(jnp.where(valid, ls, 0.0))
    out_ref[0, 0] = -total / B


def kernel(target_input, context, neg, input_emb, output_emb):
    tgt = target_input.astype(jnp.int32)
    ctx = context.astype(jnp.int32)
    negidx = neg.astype(jnp.int32).reshape(-1)
    scores = _sc_scores(tgt, ctx, negidx, input_emb, output_emb)
    loss = pl.pallas_call(
        _tc_loss_body,
        out_shape=jax.ShapeDtypeStruct((1, 1), jnp.float32),
        out_specs=pl.BlockSpec(memory_space=pltpu.SMEM),
    )(scores)
    return loss[0, 0]


# speedup vs baseline: 12.2930x; 2.3693x over previous
import functools

import jax
import jax.numpy as jnp
from jax import lax
from jax.experimental import pallas as pl
from jax.experimental.pallas import tpu as pltpu
from jax.experimental.pallas import tpu_sc as plsc

VOCAB = 1000000
DIM = 64
B = 16384
NEG = 20
NSCORE = 32
TABLE_ROWS = 2 * VOCAB
SCORE_ROWS = B * NSCORE // 128

NUM_CORES = 2
NUM_SUBCORES = 16
NW = NUM_CORES * NUM_SUBCORES
BPW = B // NW
CHUNK = 32
NCHUNK = BPW // CHUNK
NEGC = CHUNK * NEG
IDX_BLK = 128


def _sc_scores_kernel(tgt_hbm, ctx_hbm, negidx_hbm, in_emb, out_emb,
                      scores_out,
                      tgt_v0, ctx_v0, neg_v0, vrows0, urows0, nrows0,
                      tgt_v1, ctx_v1, neg_v1, vrows1, urows1, nrows1,
                      scores_s, sem0, sem1):
    wid = lax.axis_index("s") * NUM_CORES + lax.axis_index("c")
    lanes = lax.iota(jnp.int32, 16)
    slots = (
        (tgt_v0, ctx_v0, neg_v0, vrows0, urows0, nrows0, sem0),
        (tgt_v1, ctx_v1, neg_v1, vrows1, urows1, nrows1, sem1),
    )

    def fire(c, slot):
        tgt_v, ctx_v, neg_v, vrows, urows, nrows, sem = slots[slot]
        base = wid * BPW + c * CHUNK
        pltpu.sync_copy(tgt_hbm.at[pl.ds(base, CHUNK)], tgt_v)
        pltpu.sync_copy(ctx_hbm.at[pl.ds(base, CHUNK)], ctx_v)
        pltpu.sync_copy(negidx_hbm.at[pl.ds(base * NEG, NEGC)], neg_v)
        pltpu.async_copy(in_emb.at[tgt_v], vrows, sem)
        pltpu.async_copy(out_emb.at[ctx_v], urows, sem)
        for j in range(NEGC // IDX_BLK):
            pltpu.async_copy(
                out_emb.at[neg_v.at[pl.ds(j * IDX_BLK, IDX_BLK)]],
                nrows.at[pl.ds(j * IDX_BLK, IDX_BLK)], sem)

    def wait(slot):
        tgt_v, ctx_v, neg_v, vrows, urows, nrows, sem = slots[slot]
        pltpu.make_async_copy(in_emb.at[tgt_v], vrows, sem).wait()
        pltpu.make_async_copy(out_emb.at[ctx_v], urows, sem).wait()
        for j in range(NEGC // IDX_BLK):
            pltpu.make_async_copy(
                out_emb.at[neg_v.at[pl.ds(j * IDX_BLK, IDX_BLK)]],
                nrows.at[pl.ds(j * IDX_BLK, IDX_BLK)], sem).wait()

    def compute(c, slot):
        _, _, _, vrows, urows, nrows, _ = slots[slot]
        base = wid * BPW + c * CHUNK

        def b_body(b, _):
            vv = [vrows[b, pl.ds(16 * j, 16)] for j in range(4)]

            def dot_rows(rows_ref, i):
                acc = rows_ref[i, pl.ds(0, 16)] * vv[0]
                for j in range(1, 4):
                    acc = acc + rows_ref[i, pl.ds(16 * j, 16)] * vv[j]
                return jnp.sum(acc)

            vec0 = jnp.zeros((16,), jnp.float32)
            vec0 = jnp.where(lanes == 0, dot_rows(urows, b), vec0)
            for n in range(15):
                s = dot_rows(nrows, b * NEG + n)
                vec0 = jnp.where(lanes == n + 1, s, vec0)
            vec1 = jnp.zeros((16,), jnp.float32)
            for n in range(15, NEG):
                s = dot_rows(nrows, b * NEG + n)
                vec1 = jnp.where(lanes == n - 15, s, vec1)
            scores_s[b // 4, pl.ds((b % 4) * NSCORE, 16)] = vec0
            scores_s[b // 4, pl.ds((b % 4) * NSCORE + 16, 16)] = vec1
            return 0

        lax.fori_loop(0, CHUNK, b_body, 0)
        pltpu.sync_copy(
            scores_s,
            scores_out.at[pl.ds(base * NSCORE // 128, CHUNK * NSCORE // 128)])

    fire(0, 0)

    def cc_body(cc, _):
        c0 = 2 * cc
        fire(c0 + 1, 1)
        wait(0)
        compute(c0, 0)
        fire((c0 + 2) % NCHUNK, 0)
        wait(1)
        compute(c0 + 1, 1)
        return 0

    lax.fori_loop(0, NCHUNK // 2, cc_body, 0)
    wait(0)


@functools.partial(
    pl.kernel,
    mesh=plsc.VectorSubcoreMesh(core_axis_name="c", subcore_axis_name="s"),
    compiler_params=pltpu.CompilerParams(
        needs_layout_passes=False, use_tc_tiling_on_sc=False),
    out_type=jax.ShapeDtypeStruct((SCORE_ROWS, 128), jnp.float32),
    scratch_types=[
        pltpu.VMEM((CHUNK,), jnp.int32),
        pltpu.VMEM((CHUNK,), jnp.int32),
        pltpu.VMEM((NEGC,), jnp.int32),
        pltpu.VMEM((CHUNK, DIM), jnp.float32),
        pltpu.VMEM((CHUNK, DIM), jnp.float32),
        pltpu.VMEM((NEGC, DIM), jnp.float32),
        pltpu.VMEM((CHUNK,), jnp.int32),
        pltpu.VMEM((CHUNK,), jnp.int32),
        pltpu.VMEM((NEGC,), jnp.int32),
        pltpu.VMEM((CHUNK, DIM), jnp.float32),
        pltpu.VMEM((CHUNK, DIM), jnp.float32),
        pltpu.VMEM((NEGC, DIM), jnp.float32),
        pltpu.VMEM((CHUNK * NSCORE // 128, 128), jnp.float32),
        pltpu.SemaphoreType.DMA,
        pltpu.SemaphoreType.DMA,
    ],
)
def _sc_scores(*args):
    _sc_scores_kernel(*args)


TBLK = 16384


def _tc_transpose_body(x1_ref, x2_ref, o_ref):
    i0 = lax.broadcasted_iota(jnp.int32, (DIM, DIM), 0)
    i1 = lax.broadcasted_iota(jnp.int32, (DIM, DIM), 1)
    eye = jnp.where(i0 == i1, 1.0, 0.0)
    dn = (((0,), (0,)), ((), ()))
    t1 = lax.dot_general(x1_ref[...], eye, dn,
                         preferred_element_type=jnp.float32)
    t2 = lax.dot_general(x2_ref[...], eye, dn,
                         preferred_element_type=jnp.float32)
    o_ref[...] = jnp.concatenate([t1, t2], axis=1)


def _transpose_tables(in_emb_t, out_emb_t):
    grid = (pl.cdiv(VOCAB, TBLK),)
    return pl.pallas_call(
        _tc_transpose_body,
        grid=grid,
        in_specs=[
            pl.BlockSpec((DIM, TBLK), lambda i: (0, i)),
            pl.BlockSpec((DIM, TBLK), lambda i: (0, i)),
        ],
        out_specs=pl.BlockSpec((TBLK, 2 * DIM), lambda i: (i, 0)),
        out_shape=jax.ShapeDtypeStruct((VOCAB, 2 * DIM), jnp.float32),
        compiler_params=pltpu.CompilerParams(
            dimension_semantics=("arbitrary",)),
    )(in_emb_t, out_emb_t)


def _tc_loss_body(scores_ref, out_ref):
    x = scores_ref[...]
    col = lax.broadcasted_iota(jnp.int32, x.shape, 1) % NSCORE
    sign = jnp.where(col == 0, 1.0, -1.0)
    valid = col <= NEG
    ls = jax.nn.log_sigmoid(x * sign)
    total = jnp.sum(jnp.where(valid, ls, 0.0))
    out_ref[0, 0] = -total / B


def kernel(target_input, context, neg, input_emb, output_emb):
    tgt = target_input.astype(jnp.int32)
    ctx = context.astype(jnp.int32)
    negidx = neg.astype(jnp.int32).reshape(-1)
    table = _transpose_tables(input_emb.T, output_emb.T)
    table = table.reshape(TABLE_ROWS, DIM)
    scores = _sc_scores(tgt * 2, ctx * 2 + 1, negidx * 2 + 1, table, table)
    loss = pl.pallas_call(
        _tc_loss_body,
        out_shape=jax.ShapeDtypeStruct((1, 1), jnp.float32),
        out_specs=pl.BlockSpec(memory_space=pltpu.SMEM),
    )(scores)
    return loss[0, 0]
